# speedup vs baseline: 4.4841x; 4.4841x over previous
import functools

import jax
import jax.numpy as jnp
from jax import lax
from jax.experimental import pallas as pl
from jax.experimental.pallas import tpu as pltpu

D_MODEL = 1024
D_HIDDEN = 1024
N_EXP = 16
N_TOK = 2048
BLK = 128
N_BLK = 32


def _plan_kernel(x_ref, rw_ref, rb_ref, slot_ref, eob_ref, valid_ref):
    xx = x_ref[...]
    logits = jnp.dot(xx, rw_ref[...], preferred_element_type=jnp.float32)
    logits = logits + rb_ref[...]
    mx = jnp.max(logits, axis=1, keepdims=True)
    e_iota = lax.broadcasted_iota(jnp.int32, (N_TOK, N_EXP), 1)
    ids = jnp.min(jnp.where(logits == mx, e_iota, N_EXP), axis=1, keepdims=True)
    onehot = (e_iota == ids).astype(jnp.int32)

    csum = onehot
    sh = 1
    while sh < N_TOK:
        shifted = jnp.concatenate(
            [jnp.zeros((sh, N_EXP), jnp.int32), csum[: N_TOK - sh]], axis=0
        )
        csum = csum + shifted
        sh *= 2
    rank = jnp.sum(onehot * csum, axis=1, keepdims=True) - 1

    counts = jnp.sum(onehot, axis=0, keepdims=True)
    aligned = ((counts + BLK - 1) // BLK) * BLK
    acc = aligned
    sh = 1
    while sh < N_EXP:
        shifted = jnp.concatenate(
            [jnp.zeros((1, sh), jnp.int32), acc[:, : N_EXP - sh]], axis=1
        )
        acc = acc + shifted
        sh *= 2
    off = acc - aligned

    tok_off = jnp.sum(onehot * off, axis=1, keepdims=True)
    slot = tok_off + rank
    slot_ref[...] = jnp.broadcast_to(slot, (N_TOK, 128))

    total = jnp.sum(aligned, axis=1, keepdims=True)
    s0 = lax.broadcasted_iota(jnp.int32, (N_BLK, N_EXP), 0) * BLK
    eob = jnp.sum((jnp.broadcast_to(off, (N_BLK, N_EXP)) <= s0).astype(jnp.int32),
                  axis=1, keepdims=True) - 1
    valid = lax.broadcasted_iota(jnp.int32, (N_BLK, 1), 0) * BLK < total
    eob_ref[...] = jnp.broadcast_to(eob, (N_BLK, 128))
    valid_ref[...] = jnp.broadcast_to(valid.astype(jnp.int32), (N_BLK, 128))


def _moe_kernel(eob_sp, valid_sp, slot_ref, x_ref, w_ref, v_ref, out_ref):
    b = pl.program_id(0)

    @pl.when(b == 0)
    def _init():
        out_ref[...] = jnp.zeros_like(out_ref)

    @pl.when(valid_sp[b] == 1)
    def _body():
        slot = slot_ref[:, 0:1]
        r = lax.broadcasted_iota(jnp.int32, (N_TOK, BLK), 1) + b * BLK
        pt = (slot == r).astype(jnp.float32)
        xb = lax.dot_general(pt, x_ref[...], (((0,), (0,)), ((), ())),
                             preferred_element_type=jnp.float32)
        a = jnp.dot(xb, w_ref[0], preferred_element_type=jnp.float32)
        g = jnp.dot(xb, v_ref[0], preferred_element_type=jnp.float32)
        y = a * (g * (1.0 / (1.0 + jnp.exp(-g))))
        out_ref[...] += lax.dot_general(pt, y, (((1,), (0,)), ((), ())),
                                        preferred_element_type=jnp.float32)


def kernel(x, router_w, router_b, W, V):
    Bs, Ts, Dm = x.shape
    x_flat = x.reshape(Bs * Ts, Dm)
    rb = router_b.reshape(1, N_EXP)

    slot2d, eob2d, valid2d = pl.pallas_call(
        _plan_kernel,
        out_shape=[
            jax.ShapeDtypeStruct((N_TOK, 128), jnp.int32),
            jax.ShapeDtypeStruct((N_BLK, 128), jnp.int32),
            jax.ShapeDtypeStruct((N_BLK, 128), jnp.int32),
        ],
    )(x_flat, router_w, rb)

    eob = eob2d[:, 0]
    valid = valid2d[:, 0]

    grid_spec = pltpu.PrefetchScalarGridSpec(
        num_scalar_prefetch=2,
        grid=(N_BLK,),
        in_specs=[
            pl.BlockSpec((N_TOK, 128), lambda b, eob, val: (0, 0)),
            pl.BlockSpec((N_TOK, D_MODEL), lambda b, eob, val: (0, 0)),
            pl.BlockSpec((1, D_MODEL, D_HIDDEN), lambda b, eob, val: (eob[b], 0, 0)),
            pl.BlockSpec((1, D_MODEL, D_HIDDEN), lambda b, eob, val: (eob[b], 0, 0)),
        ],
        out_specs=pl.BlockSpec((N_TOK, D_HIDDEN), lambda b, eob, val: (0, 0)),
    )
    out = pl.pallas_call(
        _moe_kernel,
        grid_spec=grid_spec,
        out_shape=jax.ShapeDtypeStruct((N_TOK, D_HIDDEN), jnp.float32),
    )(eob, valid, slot2d, x_flat, W, V)
    return out.reshape(Bs, Ts, D_HIDDEN)
